# baseline (speedup 1.0000x reference)
import jax
import jax.numpy as jnp
from jax.experimental import pallas as pl
from jax.experimental.pallas import tpu as pltpu

_EPS = 1e-05
_BLOCK_ROWS = 256


def _l1norm_body(x_ref, o_ref):
    blk = x_ref[...]
    row_sum = jnp.sum(blk, axis=1, keepdims=True)
    inv = 1.0 / jnp.maximum(row_sum, _EPS)
    o_ref[...] = blk * inv


def kernel(input):
    bs, r, d = input.shape
    x = input.reshape(bs * r, d)
    n_rows = bs * r
    grid = (n_rows // _BLOCK_ROWS,)
    out = pl.pallas_call(
        _l1norm_body,
        grid=grid,
        in_specs=[pl.BlockSpec((_BLOCK_ROWS, d), lambda i: (i, 0))],
        out_specs=pl.BlockSpec((_BLOCK_ROWS, d), lambda i: (i, 0)),
        out_shape=jax.ShapeDtypeStruct((n_rows, d), x.dtype),
        compiler_params=pltpu.CompilerParams(
            dimension_semantics=("parallel",),
            vmem_limit_bytes=56 * 1024 * 1024,
        ),
        name="l1_row_norm",
    )(x)
    return out.reshape(bs, r, d)


# speedup vs baseline: 1.1232x; 1.1232x over previous
import jax
import jax.numpy as jnp
from jax.experimental import pallas as pl
from jax.experimental.pallas import tpu as pltpu

_EPS = 1e-05
_BLOCK_ROWS = 1024


def _l1norm_body(x_ref, o_ref):
    blk = x_ref[...]
    row_sum = jnp.sum(blk, axis=1, keepdims=True)
    inv = 1.0 / jnp.maximum(row_sum, _EPS)
    o_ref[...] = blk * inv


def kernel(input):
    bs, r, d = input.shape
    x = input.reshape(bs * r, d)
    n_rows = bs * r
    grid = (n_rows // _BLOCK_ROWS,)
    out = pl.pallas_call(
        _l1norm_body,
        grid=grid,
        in_specs=[pl.BlockSpec((_BLOCK_ROWS, d), lambda i: (i, 0))],
        out_specs=pl.BlockSpec((_BLOCK_ROWS, d), lambda i: (i, 0)),
        out_shape=jax.ShapeDtypeStruct((n_rows, d), x.dtype),
        compiler_params=pltpu.CompilerParams(
            dimension_semantics=("parallel",),
            vmem_limit_bytes=56 * 1024 * 1024,
        ),
        name="l1_row_norm",
    )(x)
    return out.reshape(bs, r, d)


# speedup vs baseline: 1.1285x; 1.0048x over previous
import jax
import jax.numpy as jnp
from jax.experimental import pallas as pl
from jax.experimental.pallas import tpu as pltpu

_EPS = 1e-05
_BLOCK_ROWS = 512


def _inner(x_blk, o_blk):
    blk = x_blk[...]
    row_sum = jnp.sum(blk, axis=1, keepdims=True)
    inv = 1.0 / jnp.maximum(row_sum, _EPS)
    o_blk[...] = blk * inv


def kernel(input):
    bs, r, d = input.shape
    x = input.reshape(bs * r, d)
    n_rows = bs * r
    n_blocks = n_rows // _BLOCK_ROWS

    def outer(x_hbm, o_hbm):
        pltpu.emit_pipeline(
            _inner,
            grid=(n_blocks,),
            in_specs=[
                pl.BlockSpec(
                    (_BLOCK_ROWS, d),
                    lambda i: (i, 0),
                    pipeline_mode=pl.Buffered(buffer_count=4, use_lookahead=True),
                )
            ],
            out_specs=[
                pl.BlockSpec(
                    (_BLOCK_ROWS, d),
                    lambda i: (i, 0),
                    pipeline_mode=pl.Buffered(buffer_count=2),
                )
            ],
        )(x_hbm, o_hbm)

    out = pl.pallas_call(
        outer,
        in_specs=[pl.BlockSpec(memory_space=pl.ANY)],
        out_specs=pl.BlockSpec(memory_space=pl.ANY),
        out_shape=jax.ShapeDtypeStruct((n_rows, d), x.dtype),
        compiler_params=pltpu.CompilerParams(
            vmem_limit_bytes=56 * 1024 * 1024,
        ),
        name="l1_row_norm_pipe",
    )(x)
    return out.reshape(bs, r, d)


# speedup vs baseline: 1.1558x; 1.0242x over previous
import jax
import jax.numpy as jnp
from jax.experimental import pallas as pl
from jax.experimental.pallas import tpu as pltpu

_EPS = 1e-05
_BLOCK_ROWS = 1024


def _inner(x_blk, o_blk):
    blk = x_blk[...]
    row_sum = jnp.sum(blk, axis=1, keepdims=True)
    inv = 1.0 / jnp.maximum(row_sum, _EPS)
    o_blk[...] = blk * inv


def kernel(input):
    bs, r, d = input.shape
    x = input.reshape(bs * r, d)
    n_rows = bs * r
    n_blocks = n_rows // _BLOCK_ROWS

    def outer(x_hbm, o_hbm):
        pltpu.emit_pipeline(
            _inner,
            grid=(n_blocks,),
            in_specs=[
                pl.BlockSpec(
                    (_BLOCK_ROWS, d),
                    lambda i: (i, 0),
                    pipeline_mode=pl.Buffered(buffer_count=3, use_lookahead=True),
                )
            ],
            out_specs=[
                pl.BlockSpec(
                    (_BLOCK_ROWS, d),
                    lambda i: (i, 0),
                    pipeline_mode=pl.Buffered(buffer_count=2),
                )
            ],
        )(x_hbm, o_hbm)

    out = pl.pallas_call(
        outer,
        in_specs=[pl.BlockSpec(memory_space=pl.ANY)],
        out_specs=pl.BlockSpec(memory_space=pl.ANY),
        out_shape=jax.ShapeDtypeStruct((n_rows, d), x.dtype),
        compiler_params=pltpu.CompilerParams(
            vmem_limit_bytes=56 * 1024 * 1024,
        ),
        name="l1_row_norm_pipe",
    )(x)
    return out.reshape(bs, r, d)
